# baseline (speedup 1.0000x reference)
import jax
import jax.numpy as jnp
from jax import lax
from jax.experimental import pallas as pl
from jax.experimental.pallas import tpu as pltpu
from jax.experimental.pallas import tpu_sc as plsc

_NUM_WORKERS = 32
_CHUNK = 128
_NBUF = 4
_LAG = 2
_VL = 16
_FMT_ROWS = 64


def kernel(x, table):
    batch, seq = x.shape
    _, emb = table.shape
    emb2 = 2 * emb
    n = batch * seq

    rows_per_worker = batch // _NUM_WORKERS
    per_worker = rows_per_worker * seq
    half_worker = per_worker // 2
    num_chunks = per_worker // _CHUNK
    num_groups = num_chunks // _NBUF
    hc = _CHUNK // 2
    mesh = plsc.VectorSubcoreMesh(core_axis_name="c", subcore_axis_name="s")

    @pl.kernel(
        out_type=jax.ShapeDtypeStruct((n // 2, emb2), table.dtype),
        mesh=mesh,
        compiler_params=pltpu.CompilerParams(
            use_tc_tiling_on_sc=False, needs_layout_passes=False
        ),
        scratch_types=[
            pltpu.VMEM((rows_per_worker, seq), jnp.int32),
            pltpu.VMEM((half_worker,), jnp.int32),
            pltpu.VMEM((half_worker,), jnp.int32),
            [pltpu.VMEM((hc, emb), table.dtype) for _ in range(_NBUF)],
            [pltpu.VMEM((hc, emb), table.dtype) for _ in range(_NBUF)],
            [pltpu.SemaphoreType.DMA for _ in range(_NBUF)],
            [pltpu.SemaphoreType.DMA for _ in range(_NBUF)],
        ],
    )
    def gather_kernel(table_hbm, x_hbm, out_hbm, slab, idx_e, idx_o,
                      rows_e, rows_o, gsem, wsem):
        wid = lax.axis_index("s") * 2 + lax.axis_index("c")
        rbase = wid * rows_per_worker
        base2 = wid * half_worker
        lane = lax.iota(jnp.int32, _VL)
        pltpu.sync_copy(x_hbm.at[pl.ds(rbase, rows_per_worker)], slab)

        nfull = seq // _VL
        ntail = seq - nfull * _VL
        even_mask = (lane % 2) == 0
        odd_mask = (lane % 2) == 1
        tail_col = jnp.where(lane < ntail, nfull * _VL + lane, 0)
        half = lane // 2

        @pl.loop(0, rows_per_worker)
        def _(r):
            dbase = r * (seq // 2)
            for k in range(nfull):
                v = slab[r, pl.ds(k * _VL, _VL)]
                d2 = dbase + k * (_VL // 2) + half
                plsc.store_scatter(idx_e, [d2], v, mask=even_mask)
                plsc.store_scatter(idx_o, [d2], v, mask=odd_mask)
            if ntail:
                v = plsc.load_gather(
                    slab, [jnp.full((_VL,), r, jnp.int32), tail_col],
                    mask=lane < ntail,
                )
                d2 = dbase + nfull * (_VL // 2) + half
                plsc.store_scatter(idx_e, [d2], v,
                                   mask=even_mask & (lane < ntail))
                plsc.store_scatter(idx_o, [d2], v,
                                   mask=odd_mask & (lane < ntail))

        def start_gather(c, b):
            pltpu.async_copy(
                table_hbm.at[idx_e.at[pl.ds(c * hc, hc)]], rows_e[b], gsem[b]
            )
            pltpu.async_copy(
                table_hbm.at[idx_o.at[pl.ds(c * hc, hc)]], rows_o[b], gsem[b]
            )

        def wait_gather(c, b):
            pltpu.make_async_copy(
                table_hbm.at[idx_e.at[pl.ds(c * hc, hc)]], rows_e[b], gsem[b]
            ).wait()
            pltpu.make_async_copy(
                table_hbm.at[idx_o.at[pl.ds(c * hc, hc)]], rows_o[b], gsem[b]
            ).wait()

        def start_wb(c, b):
            dst = out_hbm.at[pl.ds(base2 + c * hc, hc), pl.ds(0, emb)]
            pltpu.async_copy(rows_e[b], dst, wsem[b])
            dst2 = out_hbm.at[pl.ds(base2 + c * hc, hc), pl.ds(emb, emb)]
            pltpu.async_copy(rows_o[b], dst2, wsem[b])

        def wait_wb(c, b):
            dst = out_hbm.at[pl.ds(base2 + c * hc, hc), pl.ds(0, emb)]
            pltpu.make_async_copy(rows_e[b], dst, wsem[b]).wait()
            dst2 = out_hbm.at[pl.ds(base2 + c * hc, hc), pl.ds(emb, emb)]
            pltpu.make_async_copy(rows_o[b], dst2, wsem[b]).wait()

        for i in range(_NBUF):
            start_gather(i, i)
            if i >= _LAG:
                d = i - _LAG
                wait_gather(d, d % _NBUF)
                start_wb(d, d % _NBUF)

        @pl.loop(1, num_groups)
        def _(k):
            c0 = k * _NBUF
            for i in range(_NBUF):
                c = c0 + i
                wait_wb(c - _NBUF, i)
                start_gather(c, i)
                d = c - _LAG
                bd = (i + _NBUF - _LAG) % _NBUF
                wait_gather(d, bd)
                start_wb(d, bd)

        for d in range(num_chunks - _LAG, num_chunks):
            wait_gather(d, d % _NBUF)
            start_wb(d, d % _NBUF)
        for b in range(_NBUF):
            wait_wb(num_chunks - _NBUF + b, b)

    packed = gather_kernel(table, x)

    rows_per_step = _FMT_ROWS * seq // 2

    def fmt_body(p_ref, o_ref):
        v = p_ref[...]
        a = v[:, :emb]
        b = v[:, emb:]
        inter = jnp.stack([a, b], axis=1)
        o_ref[...] = inter.reshape(_FMT_ROWS, seq, emb)

    out = pl.pallas_call(
        fmt_body,
        grid=(batch // _FMT_ROWS,),
        in_specs=[pl.BlockSpec((rows_per_step, emb2), lambda i: (i, 0))],
        out_specs=pl.BlockSpec((_FMT_ROWS, seq, emb), lambda i: (i, 0, 0)),
        out_shape=jax.ShapeDtypeStruct((batch, seq, emb), table.dtype),
    )(packed)
    return out


# speedup vs baseline: 1.2195x; 1.2195x over previous
import jax
import jax.numpy as jnp
from jax import lax
from jax.experimental import pallas as pl
from jax.experimental.pallas import tpu as pltpu
from jax.experimental.pallas import tpu_sc as plsc

_NUM_WORKERS = 32
_CHUNK = 128
_NBUF = 4
_LAG = 2
_VL = 16


def kernel(x, table):
    batch, seq = x.shape
    _, emb = table.shape
    n = batch * seq

    rows_per_worker = batch // _NUM_WORKERS
    per_worker = rows_per_worker * seq
    num_chunks = per_worker // _CHUNK
    num_groups = num_chunks // _NBUF
    mesh = plsc.VectorSubcoreMesh(core_axis_name="c", subcore_axis_name="s")

    @pl.kernel(
        out_type=jax.ShapeDtypeStruct((n, emb), table.dtype),
        mesh=mesh,
        compiler_params=pltpu.CompilerParams(
            use_tc_tiling_on_sc=False, needs_layout_passes=False
        ),
        scratch_types=[
            pltpu.VMEM((rows_per_worker, seq), jnp.int32),
            pltpu.VMEM((per_worker,), jnp.int32),
            [pltpu.VMEM((_CHUNK, emb), table.dtype) for _ in range(_NBUF)],
            [pltpu.SemaphoreType.DMA for _ in range(_NBUF)],
            [pltpu.SemaphoreType.DMA for _ in range(_NBUF)],
        ],
    )
    def gather_kernel(table_hbm, x_hbm, out_hbm, slab, idx_flat, rows,
                      gsem, wsem):
        wid = lax.axis_index("s") * 2 + lax.axis_index("c")
        rbase = wid * rows_per_worker
        base = wid * per_worker
        lane = lax.iota(jnp.int32, _VL)
        pltpu.sync_copy(x_hbm.at[pl.ds(rbase, rows_per_worker)], slab)

        nfull = seq // _VL
        ntail = seq - nfull * _VL
        tail_col = jnp.where(lane < ntail, nfull * _VL + lane, 0)
        tail_mask = lane < ntail

        @pl.loop(0, rows_per_worker)
        def _(r):
            dbase = r * seq
            for k in range(nfull):
                v = slab[r, pl.ds(k * _VL, _VL)]
                plsc.store_scatter(idx_flat, [dbase + k * _VL + lane], v)
            if ntail:
                v = plsc.load_gather(
                    slab, [jnp.full((_VL,), r, jnp.int32), tail_col],
                    mask=tail_mask,
                )
                plsc.store_scatter(
                    idx_flat, [dbase + nfull * _VL + lane], v,
                    mask=tail_mask,
                )

        def start_gather(c, b):
            pltpu.async_copy(
                table_hbm.at[idx_flat.at[pl.ds(c * _CHUNK, _CHUNK)]],
                rows[b], gsem[b],
            )

        def wait_gather(c, b):
            pltpu.make_async_copy(
                table_hbm.at[idx_flat.at[pl.ds(c * _CHUNK, _CHUNK)]],
                rows[b], gsem[b],
            ).wait()

        def start_wb(c, b):
            pltpu.async_copy(
                rows[b], out_hbm.at[pl.ds(base + c * _CHUNK, _CHUNK)], wsem[b]
            )

        def wait_wb(c, b):
            pltpu.make_async_copy(
                rows[b], out_hbm.at[pl.ds(base + c * _CHUNK, _CHUNK)], wsem[b]
            ).wait()

        for i in range(_NBUF):
            start_gather(i, i)
            if i >= _LAG:
                d = i - _LAG
                wait_gather(d, d % _NBUF)
                start_wb(d, d % _NBUF)

        @pl.loop(1, num_groups)
        def _(k):
            c0 = k * _NBUF
            for i in range(_NBUF):
                c = c0 + i
                wait_wb(c - _NBUF, i)
                start_gather(c, i)
                d = c - _LAG
                bd = (i + _NBUF - _LAG) % _NBUF
                wait_gather(d, bd)
                start_wb(d, bd)

        for d in range(num_chunks - _LAG, num_chunks):
            wait_gather(d, d % _NBUF)
            start_wb(d, d % _NBUF)
        for b in range(_NBUF):
            wait_wb(num_chunks - _NBUF + b, b)

    out = gather_kernel(table, x)
    return out.reshape(batch, seq, emb)
